# Initial kernel scaffold; baseline (speedup 1.0000x reference)
import jax
import jax.numpy as jnp
from jax.experimental import pallas as pl


def kernel(context_vector, center, tractovki):
    raise NotImplementedError("write your pallas kernel here")


# speedup vs baseline: 1.2012x; 1.2012x over previous
import functools

import jax
import jax.numpy as jnp
from jax import lax
from jax.experimental import pallas as pl
from jax.experimental.pallas import tpu as pltpu
from jax.experimental.pallas import tpu_sc as plsc

N = 100000
D = 128
NSEG = D // 16
NC = 2
NS = 16
NW = NC * NS
RPW = -(-N // NW // 8) * 8
LAST_START = N - RPW
CH = 128
NFULL = RPW // CH
TAIL = RPW - NFULL * CH
TAIL_GROUPS = (TAIL + 15) // 16

_mesh = plsc.VectorSubcoreMesh(
    core_axis_name="c", subcore_axis_name="s", num_cores=NC, num_subcores=NS)

_params = pltpu.CompilerParams(needs_layout_passes=False)

_NEG_INF = float("-inf")
_IMAX = 2**31 - 1


def _row_key(buf, row, dsegs):
  acc_d = jnp.zeros((16,), jnp.float32)
  acc_n = jnp.zeros((16,), jnp.float32)
  for k in range(NSEG):
    v = buf[row, pl.ds(16 * k, 16)]
    acc_d = acc_d + v * dsegs[k]
    acc_n = acc_n + v * v
  return jnp.sum(acc_d), jnp.sum(acc_n)


def _process_chunk(buf, base, limit, dsegs, lane, runk, runi, ngroups):
  def group_body(g, carry):
    runk, runi = carry

    def quad_body(q, kc):
      kd, kn = kc
      for rr in range(4):
        r = q * 4 + rr
        dot, nsq = _row_key(buf, g * 16 + r, dsegs)
        m = lane == r
        kd = jnp.where(m, dot, kd)
        kn = jnp.where(m, nsq, kn)
      return kd, kn

    zero = jnp.zeros((16,), jnp.float32)
    kd, kn = lax.fori_loop(0, 4, quad_body, (zero, zero))
    key = kd * jnp.abs(kd) / jnp.maximum(kn, jnp.float32(1e-30))
    gidx = base + g * 16 + lane
    key = jnp.where(gidx < limit, key, _NEG_INF)
    upd = key > runk
    runi = jnp.where(upd, gidx, runi)
    runk = jnp.where(upd, key, runk)
    return runk, runi

  return lax.fori_loop(0, ngroups, group_body, (runk, runi))


def _stage1_body(ctx_h, cen_h, tract_h, keys_h, idxs_h,
                 ctx_v, cen_v, buf0, buf1, kout_v, iout_v, sem0, sem1):
  c = lax.axis_index("c")
  s = lax.axis_index("s")
  wid = s * NC + c
  start = pl.multiple_of(jnp.minimum(wid * RPW, LAST_START), 8)
  limit = start + RPW

  pltpu.sync_copy(ctx_h, ctx_v)
  pltpu.sync_copy(cen_h, cen_v)
  dsegs = [ctx_v[pl.ds(16 * k, 16)] - cen_v[pl.ds(16 * k, 16)]
           for k in range(NSEG)]
  lane = lax.iota(jnp.int32, 16)

  bufs = (buf0, buf1)
  sems = (sem0, sem1)

  def full_copy(g, b):
    return pltpu.make_async_copy(
        tract_h.at[pl.ds(start + g * CH, CH)], bufs[b], sems[b])

  def tail_copy():
    return pltpu.make_async_copy(
        tract_h.at[pl.ds(start + NFULL * CH, TAIL)],
        buf0.at[pl.ds(0, TAIL)], sem0)

  full_copy(0, 0).start()
  full_copy(1, 1).start()

  runk = jnp.full((16,), _NEG_INF, jnp.float32)
  runi = jnp.zeros((16,), jnp.int32)

  def pair_body(p, carry):
    runk, runi = carry
    for b in range(2):
      g = 2 * p + b
      full_copy(g, b).wait()
      runk, runi = _process_chunk(
          bufs[b], start + g * CH, limit, dsegs, lane, runk, runi, 8)
      full_copy(g + 2, b).start()
    return runk, runi

  runk, runi = lax.fori_loop(0, NFULL // 2 - 1, pair_body, (runk, runi))

  full_copy(NFULL - 2, 0).wait()
  runk, runi = _process_chunk(
      buf0, start + (NFULL - 2) * CH, limit, dsegs, lane, runk, runi, 8)
  tail_copy().start()
  full_copy(NFULL - 1, 1).wait()
  runk, runi = _process_chunk(
      buf1, start + (NFULL - 1) * CH, limit, dsegs, lane, runk, runi, 8)
  tail_copy().wait()
  runk, runi = _process_chunk(
      buf0, start + NFULL * CH, limit, dsegs, lane, runk, runi, TAIL_GROUPS)

  m = jnp.max(runk)
  cand = jnp.where(runk == m, runi, _IMAX)
  bi = jnp.min(cand)
  for i in range(8):
    kout_v[i, :] = jnp.zeros((16,), jnp.float32) + m
    iout_v[i, :] = jnp.zeros((16,), jnp.int32) + bi
  off = pl.multiple_of(wid * 8, 8)
  pltpu.sync_copy(kout_v, keys_h.at[pl.ds(off, 8)])
  pltpu.sync_copy(iout_v, idxs_h.at[pl.ds(off, 8)])


_stage1 = pl.kernel(
    _stage1_body,
    out_type=(
        jax.ShapeDtypeStruct((NW * 8, 16), jnp.float32),
        jax.ShapeDtypeStruct((NW * 8, 16), jnp.int32),
    ),
    mesh=_mesh,
    compiler_params=_params,
    scratch_types=[
        pltpu.VMEM((D,), jnp.float32),
        pltpu.VMEM((D,), jnp.float32),
        pltpu.VMEM((CH, D), jnp.float32),
        pltpu.VMEM((CH, D), jnp.float32),
        pltpu.VMEM((8, 16), jnp.float32),
        pltpu.VMEM((8, 16), jnp.int32),
        pltpu.SemaphoreType.DMA,
        pltpu.SemaphoreType.DMA,
    ],
)


def _merge_body(keys_ref, idxs_ref, tract_ref, row_ref, bi_ref, ci_ref,
                rows_v, bs_s, sem):
  kmat = keys_ref[...]
  imat = idxs_ref[...]
  m = jnp.max(kmat)
  cand = jnp.where(kmat == m, imat, _IMAX)
  best = jnp.min(cand)
  bs_s[0] = best
  best_s = bs_s[0]
  base8 = pl.multiple_of((best_s // 8) * 8, 8)
  cp = pltpu.make_async_copy(tract_ref.at[pl.ds(base8, 8)], rows_v, sem)
  cp.start()
  cp.wait()
  r = best_s - base8
  row_ref[...] = rows_v[pl.ds(r, 1), :]
  bi_ref[...] = jnp.full((1, 1), best, jnp.int32)
  ci_ref[...] = (jnp.full((1, 1), best, jnp.int32).astype(jnp.float32)
                 / jnp.float32(100.0)).astype(jnp.int32)


_merge_tc = pl.pallas_call(
    _merge_body,
    out_shape=(
        jax.ShapeDtypeStruct((1, D), jnp.float32),
        jax.ShapeDtypeStruct((1, 1), jnp.int32),
        jax.ShapeDtypeStruct((1, 1), jnp.int32),
    ),
    in_specs=[
        pl.BlockSpec(memory_space=pltpu.VMEM),
        pl.BlockSpec(memory_space=pltpu.VMEM),
        pl.BlockSpec(memory_space=pl.ANY),
    ],
    scratch_shapes=[
        pltpu.VMEM((8, D), jnp.float32),
        pltpu.SMEM((1,), jnp.int32),
        pltpu.SemaphoreType.DMA,
    ],
)


@jax.jit
def kernel(context_vector, center, tractovki):
  keys, idxs = _stage1(context_vector, center, tractovki)
  row, besti, ctxi = _merge_tc(keys[::8], idxs[::8], tractovki)
  return row[0], besti[0, 0], ctxi[0, 0]
